# baseline (speedup 1.0000x reference)
import functools

import jax
import jax.numpy as jnp
from jax import lax
from jax.experimental import pallas as pl
from jax.experimental.pallas import tpu as pltpu
from jax.experimental.pallas import tpu_sc as plsc

B = 4096
F = 26
D = 32
DD = 13
NROW = 1000000
NW = 32
CHUNK = 128
NCH = B // CHUNK
NPAIR = F * D
PAIR_PER_W = NPAIR // NW
NOUT = (F + DD) * D


@functools.lru_cache(maxsize=None)
def _get_sc_gather():
    mesh = plsc.VectorSubcoreMesh(core_axis_name="c", subcore_axis_name="s")

    @functools.partial(
        pl.kernel,
        mesh=mesh,
        out_type=jax.ShapeDtypeStruct((NOUT, B), jnp.float32),
        scratch_types=[
            pltpu.VMEM((2, NCH, CHUNK), jnp.int32),
            pltpu.VMEM((PAIR_PER_W, B), jnp.float32),
            pltpu.SemaphoreType.DMA,
        ],
        compiler_params=pltpu.CompilerParams(use_tc_tiling_on_sc=False),
    )
    def _sc_gather(tablet_hbm, idx_hbm, out_hbm, idx_v, cols_v, sem):
        wid = lax.axis_index("s") * 2 + lax.axis_index("c")
        p0 = wid * PAIR_PER_W
        f_lo = p0 // D
        f_hi = (p0 + PAIR_PER_W - 1) // D
        pltpu.sync_copy(idx_hbm.at[f_lo], idx_v.at[0])

        @pl.when(f_hi != f_lo)
        def _():
            pltpu.sync_copy(idx_hbm.at[f_hi], idx_v.at[1])

        def body(q, carry):
            p = p0 + q
            f = p // D
            slot = f - f_lo
            d = p - f * D
            col = tablet_hbm.at[d]
            for c in range(NCH):
                pltpu.async_copy(
                    col.at[idx_v.at[slot].at[c]],
                    cols_v.at[q].at[pl.ds(c * CHUNK, CHUNK)],
                    sem,
                )
            @pl.when(q >= 1)
            def _():
                pltpu.make_async_copy(out_hbm.at[0], cols_v.at[0], sem).wait()
            return carry

        lax.fori_loop(0, PAIR_PER_W, body, 0)
        pltpu.make_async_copy(out_hbm.at[0], cols_v.at[0], sem).wait()
        pltpu.sync_copy(cols_v, out_hbm.at[pl.ds(p0, PAIR_PER_W)])

    return _sc_gather


def _tc_dense(xt_ref, wt_ref, b_ref, alias_ref, out_ref):
    del alias_ref
    acc = jax.lax.dot_general(
        wt_ref[...], xt_ref[...],
        (((0,), (0,)), ((), ())),
        preferred_element_type=jnp.float32,
    )
    out_ref[...] = acc + b_ref[...]


def kernel(sparse_inputs, dense_inputs, table, W, b):
    tablet = table.T
    idxc = sparse_inputs.T.astype(jnp.int32).reshape(F, NCH, CHUNK)
    out_t = _get_sc_gather()(tablet, idxc)
    xt = dense_inputs.T
    wt = W.T
    out_t = pl.pallas_call(
        _tc_dense,
        grid=(1,),
        in_specs=[
            pl.BlockSpec((DD, B), lambda i: (0, 0)),
            pl.BlockSpec((DD, D * DD), lambda i: (0, 0)),
            pl.BlockSpec((D * DD, 1), lambda i: (0, 0)),
            pl.BlockSpec(memory_space=pl.ANY),
        ],
        out_specs=pl.BlockSpec((D * DD, B), lambda i: (2, 0)),
        out_shape=jax.ShapeDtypeStruct((NOUT, B), jnp.float32),
        input_output_aliases={3: 0},
    )(xt, wt, b.reshape(D * DD, 1), out_t)
    return out_t.reshape(F + DD, D, B).transpose(2, 0, 1)


# speedup vs baseline: 4.6787x; 4.6787x over previous
import functools

import jax
import jax.numpy as jnp
from jax import lax
from jax.experimental import pallas as pl
from jax.experimental.pallas import tpu as pltpu
from jax.experimental.pallas import tpu_sc as plsc

B = 4096
F = 26
D = 32
DD = 13
NW = 32
CHUNK = 128
NIDX = B * F
IDX_PER_W = NIDX // NW
NCHUNK = IDX_PER_W // CHUNK
NOUT = (F + DD) * D


@functools.lru_cache(maxsize=None)
def _get_sc_gather():
    mesh = plsc.VectorSubcoreMesh(core_axis_name="c", subcore_axis_name="s")

    @functools.partial(
        pl.kernel,
        mesh=mesh,
        out_type=jax.ShapeDtypeStruct((NIDX, D), jnp.float32),
        scratch_types=[
            pltpu.VMEM((NCHUNK, CHUNK), jnp.int32),
            pltpu.VMEM((IDX_PER_W, D), jnp.float32),
            pltpu.SemaphoreType.DMA,
        ],
        compiler_params=pltpu.CompilerParams(use_tc_tiling_on_sc=False),
    )
    def _sc_gather(table_hbm, idx_hbm, out_hbm, idx_v, rows_v, sem):
        wid = lax.axis_index("s") * 2 + lax.axis_index("c")
        pltpu.sync_copy(idx_hbm.at[wid], idx_v)
        copies = []
        for j in range(NCHUNK):
            copies.append(
                pltpu.async_copy(
                    table_hbm.at[idx_v.at[j]],
                    rows_v.at[pl.ds(j * CHUNK, CHUNK)],
                    sem,
                )
            )
        for c in copies:
            c.wait()
        pltpu.sync_copy(rows_v, out_hbm.at[pl.ds(wid * IDX_PER_W, IDX_PER_W)])

    return _sc_gather


BB = 512


def _tc_assemble(g_ref, xt_ref, wt_ref, b_ref, out_ref):
    for f in range(F):
        out_ref[pl.ds(f * D, D), :] = g_ref[f].T
    acc = jax.lax.dot_general(
        wt_ref[...], xt_ref[...],
        (((0,), (0,)), ((), ())),
        preferred_element_type=jnp.float32,
    )
    out_ref[pl.ds(F * D, DD * D), :] = acc + b_ref[...]


def kernel(sparse_inputs, dense_inputs, table, W, b):
    idxc = sparse_inputs.T.astype(jnp.int32).reshape(NW, NCHUNK, CHUNK)
    g = _get_sc_gather()(table, idxc)
    g3 = g.reshape(F, B, D)
    xt = dense_inputs.T
    wt = W.T
    out_t = pl.pallas_call(
        _tc_assemble,
        grid=(B // BB,),
        in_specs=[
            pl.BlockSpec((F, BB, D), lambda i: (0, i, 0)),
            pl.BlockSpec((DD, BB), lambda i: (0, i)),
            pl.BlockSpec((DD, D * DD), lambda i: (0, 0)),
            pl.BlockSpec((D * DD, 1), lambda i: (0, 0)),
        ],
        out_specs=pl.BlockSpec((NOUT, BB), lambda i: (0, i)),
        out_shape=jax.ShapeDtypeStruct((NOUT, B), jnp.float32),
    )(g3, xt, wt, b.reshape(D * DD, 1))
    return out_t.reshape(F + DD, D, B).transpose(2, 0, 1)
